# baseline (speedup 1.0000x reference)
import functools

import jax
import jax.numpy as jnp
from jax import lax
from jax.experimental import pallas as pl
from jax.experimental.pallas import tpu as pltpu
from jax.experimental.pallas import tpu_sc as plsc

D = 128
CHUNK = 128
NC = 2
NS = 16
NW = NC * NS


def _make_sc_agg(n_pad, n_chunks, with_deg):
  assert n_chunks % 4 == 0 and n_chunks >= 8
  n_quads = n_chunks // 4
  rows_per_tile = n_pad // NS
  zero_blocks = rows_per_tile // CHUNK
  mesh = plsc.VectorSubcoreMesh(core_axis_name="c", subcore_axis_name="s")

  out_type = [jax.ShapeDtypeStruct((NC, n_pad, D), jnp.float32)]
  if with_deg:
    out_type.append(jax.ShapeDtypeStruct((NC, n_pad), jnp.float32))

  scratch = [
      pltpu.VMEM((8, CHUNK), jnp.int32),
      pltpu.VMEM((8, CHUNK), jnp.int32),
      pltpu.VMEM((CHUNK, D), jnp.float32),
      pltpu.VMEM((CHUNK, D), jnp.float32),
      pltpu.VMEM((CHUNK,), jnp.float32),
      pltpu.VMEM_SHARED((n_pad, D), jnp.float32),
      pltpu.VMEM_SHARED((n_pad,), jnp.float32),
      pltpu.SemaphoreType.DMA,
      pltpu.SemaphoreType.DMA,
      pltpu.SemaphoreType.DMA,
      pltpu.SemaphoreType.DMA,
      pltpu.SemaphoreType.DMA,
  ]

  @functools.partial(
      pl.kernel,
      mesh=mesh,
      out_type=tuple(out_type),
      scratch_types=scratch,
  )
  def sc_agg(x_hbm, idx_hbm, *refs):
    if with_deg:
      sum_out, deg_out = refs[0], refs[1]
      rest = refs[2:]
    else:
      sum_out = refs[0]
      deg_out = None
      rest = refs[1:]
    (ib0, ib1, rows_a, rows_b, ones_v, acc_sh, deg_sh,
     g_a, g_b, s_a, s_b, i_sem) = rest
    ibufs = (ib0, ib1)
    rows = (rows_a, rows_b)
    gsems = (g_a, g_b)
    ssems = (s_a, s_b)

    c = lax.axis_index("c")
    s = lax.axis_index("s")
    wid = s * NC + c
    base = s * rows_per_tile

    def zfill(i, _):
      rows_a[i // (D // 16), pl.ds((i % (D // 16)) * 16, 16)] = (
          jnp.zeros((16,), jnp.float32))
      return 0
    lax.fori_loop(0, CHUNK * (D // 16), zfill, 0)
    if with_deg:
      for i in range(CHUNK // 16):
        ones_v[pl.ds(i * 16, 16)] = jnp.ones((16,), jnp.float32)

    for k in range(zero_blocks):
      pltpu.sync_copy(rows_a, acc_sh.at[pl.ds(base + k * CHUNK, CHUNK)])
    if with_deg:
      for k in range(zero_blocks):
        pltpu.sync_copy(rows_a.at[0], deg_sh.at[pl.ds(base + k * CHUNK, CHUNK)])
    plsc.subcore_barrier()

    pltpu.sync_copy(idx_hbm.at[wid, pl.ds(0, 8)], ib0)
    pltpu.async_copy(x_hbm.at[ib0.at[0]], rows_a, g_a)

    def quad(q, _):
      iq = q % 2
      def do_quad(i_cur, i_nxt):
        for j in range(4):
          ch = 4 * q + j
          b, o = rows[j % 2], rows[(j + 1) % 2]
          gb, go = gsems[j % 2], gsems[(j + 1) % 2]
          sb, so = ssems[j % 2], ssems[(j + 1) % 2]
          pltpu.make_async_copy(x_hbm.at[i_cur.at[2 * j]], b, gb).wait()
          if j == 0:
            @pl.when(q > 0)
            def _():
              pltpu.make_async_copy(o, acc_sh.at[i_nxt.at[7]], so).wait()
              if with_deg:
                pltpu.make_async_copy(ones_v, deg_sh.at[i_nxt.at[7]],
                                      so).wait()
            @pl.when(q + 1 < n_quads)
            def _():
              pltpu.async_copy(idx_hbm.at[wid, pl.ds(8 * (q + 1), 8)],
                               i_nxt, i_sem)
          else:
            pltpu.make_async_copy(o, acc_sh.at[i_cur.at[2 * j - 1]],
                                  so).wait()
            if with_deg:
              pltpu.make_async_copy(ones_v, deg_sh.at[i_cur.at[2 * j - 1]],
                                    so).wait()
          if j < 3:
            pltpu.async_copy(x_hbm.at[i_cur.at[2 * (j + 1)]], o, go)
          pltpu.async_copy(b, acc_sh.at[i_cur.at[2 * j + 1]], sb, add=True)
          if with_deg:
            pltpu.async_copy(ones_v, deg_sh.at[i_cur.at[2 * j + 1]], sb,
                             add=True)
        @pl.when(q + 1 < n_quads)
        def _():
          pltpu.make_async_copy(idx_hbm.at[wid, pl.ds(8 * (q + 1), 8)],
                                i_nxt, i_sem).wait()
          pltpu.async_copy(x_hbm.at[i_nxt.at[0]], rows_a, g_a)

      @pl.when(iq == 0)
      def _():
        do_quad(ibufs[0], ibufs[1])
      @pl.when(iq == 1)
      def _():
        do_quad(ibufs[1], ibufs[0])
      return 0

    lax.fori_loop(0, n_quads, quad, 0)

    last_ib = ibufs[(n_quads - 1) % 2]
    pltpu.make_async_copy(rows_b, acc_sh.at[last_ib.at[7]], s_b).wait()
    if with_deg:
      pltpu.make_async_copy(ones_v, deg_sh.at[last_ib.at[7]], s_b).wait()

    plsc.subcore_barrier()
    pltpu.sync_copy(acc_sh.at[pl.ds(base, rows_per_tile)],
                    sum_out.at[c, pl.ds(base, rows_per_tile)])
    if with_deg:
      pltpu.sync_copy(deg_sh.at[pl.ds(base, rows_per_tile)],
                      deg_out.at[c, pl.ds(base, rows_per_tile)])

  return sc_agg


def _dot(a, b):
  return jnp.dot(a, b, precision=lax.Precision.HIGHEST,
                 preferred_element_type=jnp.float32)


def _leaky(h):
  return jnp.where(h >= 0, h, 0.01 * h)


def _tc_layer1(sums, deg, xp, Wl, bl, Wr, n_pad, br=512):
  def body(sum_ref, deg_ref, x_ref, wl_ref, bl_ref, wr_ref, h_ref):
    r = pl.program_id(0)
    s = sum_ref[0] + sum_ref[1]
    dg = deg_ref[:, pl.ds(r * br, br)]
    dg = jnp.clip(dg[0] + dg[1], 1.0, None)
    mean = s / dg[:, None]
    h = _dot(mean, wl_ref[...]) + bl_ref[...] + _dot(x_ref[...], wr_ref[...])
    norm = jnp.sqrt(jnp.sum(h * h, axis=1, keepdims=True))
    h = h / jnp.clip(norm, 1e-12, None)
    h_ref[...] = _leaky(h)

  return pl.pallas_call(
      body,
      grid=(n_pad // br,),
      in_specs=[
          pl.BlockSpec((NC, br, D), lambda r: (0, r, 0)),
          pl.BlockSpec((NC, n_pad), lambda r: (0, 0)),
          pl.BlockSpec((br, D), lambda r: (r, 0)),
          pl.BlockSpec((D, D), lambda r: (0, 0)),
          pl.BlockSpec((1, D), lambda r: (0, 0)),
          pl.BlockSpec((D, D), lambda r: (0, 0)),
      ],
      out_specs=pl.BlockSpec((br, D), lambda r: (r, 0)),
      out_shape=jax.ShapeDtypeStruct((n_pad, D), jnp.float32),
  )(sums, deg, xp, Wl, bl, Wr)


def _tc_layer2(sums, deg, hp, Wl, bl, Wr, Wlin, blin, n_pad, br=512):
  def body(sum_ref, deg_ref, h_ref, wl_ref, bl_ref, wr_ref, wlin_ref,
           blin_ref, out_ref):
    r = pl.program_id(0)
    s = sum_ref[0] + sum_ref[1]
    dg = deg_ref[:, pl.ds(r * br, br)]
    dg = jnp.clip(dg[0] + dg[1], 1.0, None)
    mean = s / dg[:, None]
    h = _dot(mean, wl_ref[...]) + bl_ref[...] + _dot(h_ref[...], wr_ref[...])
    h = _leaky(h)
    out_ref[...] = _dot(h, wlin_ref[...]) + blin_ref[...]

  return pl.pallas_call(
      body,
      grid=(n_pad // br,),
      in_specs=[
          pl.BlockSpec((NC, br, D), lambda r: (0, r, 0)),
          pl.BlockSpec((NC, n_pad), lambda r: (0, 0)),
          pl.BlockSpec((br, D), lambda r: (r, 0)),
          pl.BlockSpec((D, D), lambda r: (0, 0)),
          pl.BlockSpec((1, D), lambda r: (0, 0)),
          pl.BlockSpec((D, D), lambda r: (0, 0)),
          pl.BlockSpec((D, 1), lambda r: (0, 0)),
          pl.BlockSpec((1, 1), lambda r: (0, 0)),
      ],
      out_specs=pl.BlockSpec((br, 1), lambda r: (r, 0)),
      out_shape=jax.ShapeDtypeStruct((n_pad, 1), jnp.float32),
  )(sums, deg, hp, Wl, bl, Wr, Wlin, blin)


def kernel(x, edge_index, edge_weight, Wl1, bl1, Wr1, Wl2, bl2, Wr2,
           Wlin, blin):
  del edge_weight
  n = x.shape[0]
  e = edge_index.shape[1]

  rows_per_tile = -(-(n + 1) // (NS * CHUNK)) * CHUNK
  n_pad = NS * rows_per_tile

  n_chunks = -(-(-(-e // (NW * CHUNK))) // 4) * 4
  e_pad = NW * n_chunks * CHUNK

  src = jnp.concatenate(
      [edge_index[0], jnp.zeros((e_pad - e,), jnp.int32)]).reshape(
          NW, n_chunks, CHUNK)
  dst = jnp.concatenate(
      [edge_index[1], jnp.full((e_pad - e,), n, jnp.int32)]).reshape(
          NW, n_chunks, CHUNK)
  idx = jnp.stack([src, dst], axis=2).reshape(NW, 2 * n_chunks, CHUNK)

  xp = jnp.pad(x, ((0, n_pad - n), (0, 0)))

  sc_agg1 = _make_sc_agg(n_pad, n_chunks, with_deg=True)
  sc_agg2 = _make_sc_agg(n_pad, n_chunks, with_deg=False)

  sums1, deg = sc_agg1(xp, idx)
  h1 = _tc_layer1(sums1, deg, xp, Wl1, bl1.reshape(1, D), Wr1, n_pad)
  (sums2,) = sc_agg2(h1, idx)
  out = _tc_layer2(sums2, deg, h1, Wl2, bl2.reshape(1, D), Wr2,
                   Wlin, blin.reshape(1, 1), n_pad)
  return out[:n]


# speedup vs baseline: 1.1240x; 1.1240x over previous
import functools

import jax
import jax.numpy as jnp
from jax import lax
from jax.experimental import pallas as pl
from jax.experimental.pallas import tpu as pltpu
from jax.experimental.pallas import tpu_sc as plsc

D = 128
CHUNK = 128
NC = 2
NS = 16
NW = NC * NS


def _make_sc_agg(n_pad, n_chunks, with_deg):
  rows_per_tile = n_pad // NS
  zero_blocks = rows_per_tile // CHUNK
  mesh = plsc.VectorSubcoreMesh(core_axis_name="c", subcore_axis_name="s")

  out_type = [jax.ShapeDtypeStruct((NC, n_pad, D), jnp.float32)]
  if with_deg:
    out_type.append(jax.ShapeDtypeStruct((NC, n_pad), jnp.float32))

  scratch = [
      pltpu.VMEM((n_chunks, CHUNK), jnp.int32),
      pltpu.VMEM((n_chunks, CHUNK), jnp.int32),
      pltpu.VMEM((CHUNK, D), jnp.float32),
      pltpu.VMEM((CHUNK,), jnp.float32),
      pltpu.VMEM_SHARED((n_pad, D), jnp.float32),
      pltpu.VMEM_SHARED((n_pad,), jnp.float32),
      pltpu.SemaphoreType.DMA,
  ]

  @functools.partial(
      pl.kernel,
      mesh=mesh,
      out_type=tuple(out_type),
      scratch_types=scratch,
  )
  def sc_agg(x_hbm, src_hbm, dst_hbm, *refs):
    if with_deg:
      sum_out, deg_out = refs[0], refs[1]
      rest = refs[2:]
    else:
      sum_out = refs[0]
      deg_out = None
      rest = refs[1:]
    src_v, dst_v, rows_v, ones_v, acc_sh, deg_sh, sem = rest

    c = lax.axis_index("c")
    s = lax.axis_index("s")
    wid = s * NC + c
    base = s * rows_per_tile

    def zfill(i, _):
      rows_v[i // (D // 16), pl.ds((i % (D // 16)) * 16, 16)] = (
          jnp.zeros((16,), jnp.float32))
      return 0
    lax.fori_loop(0, CHUNK * (D // 16), zfill, 0)
    for i in range(CHUNK // 16):
      ones_v[pl.ds(i * 16, 16)] = jnp.ones((16,), jnp.float32)

    for k in range(zero_blocks):
      pltpu.sync_copy(rows_v, acc_sh.at[pl.ds(base + k * CHUNK, CHUNK)])
    if with_deg:
      for k in range(zero_blocks):
        pltpu.sync_copy(rows_v.at[0], deg_sh.at[pl.ds(base + k * CHUNK, CHUNK)])
    plsc.subcore_barrier()

    pltpu.sync_copy(src_hbm.at[wid], src_v)
    pltpu.sync_copy(dst_hbm.at[wid], dst_v)

    def body(i, _):
      pltpu.async_copy(x_hbm.at[src_v.at[i]], rows_v, sem).wait()
      pltpu.sync_copy(rows_v, acc_sh.at[dst_v.at[i]], add=True)
      if with_deg:
        pltpu.sync_copy(ones_v, deg_sh.at[dst_v.at[i]], add=True)
      return 0
    lax.fori_loop(0, n_chunks, body, 0)

    plsc.subcore_barrier()
    pltpu.sync_copy(acc_sh.at[pl.ds(base, rows_per_tile)],
                    sum_out.at[c, pl.ds(base, rows_per_tile)])
    if with_deg:
      pltpu.sync_copy(deg_sh.at[pl.ds(base, rows_per_tile)],
                      deg_out.at[c, pl.ds(base, rows_per_tile)])

  return sc_agg


def _dot(a, b):
  return jnp.dot(a, b, precision=lax.Precision.HIGHEST,
                 preferred_element_type=jnp.float32)


def _leaky(h):
  return jnp.where(h >= 0, h, 0.01 * h)


def _tc_layer1(sums, deg, xp, Wl, bl, Wr, n_pad, br=512):
  def body(sum_ref, deg_ref, x_ref, wl_ref, bl_ref, wr_ref, h_ref):
    r = pl.program_id(0)
    s = sum_ref[0] + sum_ref[1]
    dg = deg_ref[:, pl.ds(r * br, br)]
    dg = jnp.clip(dg[0] + dg[1], 1.0, None)
    mean = s / dg[:, None]
    h = _dot(mean, wl_ref[...]) + bl_ref[...] + _dot(x_ref[...], wr_ref[...])
    norm = jnp.sqrt(jnp.sum(h * h, axis=1, keepdims=True))
    h = h / jnp.clip(norm, 1e-12, None)
    h_ref[...] = _leaky(h)

  return pl.pallas_call(
      body,
      grid=(n_pad // br,),
      in_specs=[
          pl.BlockSpec((NC, br, D), lambda r: (0, r, 0)),
          pl.BlockSpec((NC, n_pad), lambda r: (0, 0)),
          pl.BlockSpec((br, D), lambda r: (r, 0)),
          pl.BlockSpec((D, D), lambda r: (0, 0)),
          pl.BlockSpec((1, D), lambda r: (0, 0)),
          pl.BlockSpec((D, D), lambda r: (0, 0)),
      ],
      out_specs=pl.BlockSpec((br, D), lambda r: (r, 0)),
      out_shape=jax.ShapeDtypeStruct((n_pad, D), jnp.float32),
  )(sums, deg, xp, Wl, bl, Wr)


def _tc_layer2(sums, deg, hp, Wl, bl, Wr, Wlin, blin, n_pad, br=512):
  def body(sum_ref, deg_ref, h_ref, wl_ref, bl_ref, wr_ref, wlin_ref,
           blin_ref, out_ref):
    r = pl.program_id(0)
    s = sum_ref[0] + sum_ref[1]
    dg = deg_ref[:, pl.ds(r * br, br)]
    dg = jnp.clip(dg[0] + dg[1], 1.0, None)
    mean = s / dg[:, None]
    h = _dot(mean, wl_ref[...]) + bl_ref[...] + _dot(h_ref[...], wr_ref[...])
    h = _leaky(h)
    out_ref[...] = _dot(h, wlin_ref[...]) + blin_ref[...]

  return pl.pallas_call(
      body,
      grid=(n_pad // br,),
      in_specs=[
          pl.BlockSpec((NC, br, D), lambda r: (0, r, 0)),
          pl.BlockSpec((NC, n_pad), lambda r: (0, 0)),
          pl.BlockSpec((br, D), lambda r: (r, 0)),
          pl.BlockSpec((D, D), lambda r: (0, 0)),
          pl.BlockSpec((1, D), lambda r: (0, 0)),
          pl.BlockSpec((D, D), lambda r: (0, 0)),
          pl.BlockSpec((D, 1), lambda r: (0, 0)),
          pl.BlockSpec((1, 1), lambda r: (0, 0)),
      ],
      out_specs=pl.BlockSpec((br, 1), lambda r: (r, 0)),
      out_shape=jax.ShapeDtypeStruct((n_pad, 1), jnp.float32),
  )(sums, deg, hp, Wl, bl, Wr, Wlin, blin)


def kernel(x, edge_index, edge_weight, Wl1, bl1, Wr1, Wl2, bl2, Wr2,
           Wlin, blin):
  del edge_weight
  n = x.shape[0]
  e = edge_index.shape[1]

  rows_per_tile = -(-(n + 1) // (NS * CHUNK)) * CHUNK
  n_pad = NS * rows_per_tile

  n_chunks = -(-e // (NW * CHUNK))
  e_pad = NW * n_chunks * CHUNK

  src = jnp.concatenate(
      [edge_index[0], jnp.zeros((e_pad - e,), jnp.int32)]).reshape(
          NW, n_chunks, CHUNK)
  pad_dst = n + jnp.arange(e_pad - e, dtype=jnp.int32) % (n_pad - n)
  dst = jnp.concatenate([edge_index[1], pad_dst]).reshape(
      NW, n_chunks, CHUNK)

  xp = jnp.pad(x, ((0, n_pad - n), (0, 0)))

  sc_agg1 = _make_sc_agg(n_pad, n_chunks, with_deg=True)
  sc_agg2 = _make_sc_agg(n_pad, n_chunks, with_deg=False)

  sums1, deg = sc_agg1(xp, src, dst)
  h1 = _tc_layer1(sums1, deg, xp, Wl1, bl1.reshape(1, D), Wr1, n_pad)
  (sums2,) = sc_agg2(h1, src, dst)
  out = _tc_layer2(sums2, deg, h1, Wl2, bl2.reshape(1, D), Wr2,
                   Wlin, blin.reshape(1, 1), n_pad)
  return out[:n]


# speedup vs baseline: 1.5616x; 1.3893x over previous
import functools

import jax
import jax.numpy as jnp
from jax import lax
from jax.experimental import pallas as pl
from jax.experimental.pallas import tpu as pltpu
from jax.experimental.pallas import tpu_sc as plsc

D = 128
CHUNK = 128
NC = 2
NS = 16
NW = NC * NS


def _make_sc_agg(n_pad, k0, k1, with_deg):
  n_chunks = max(k0, k1)
  rows_per_tile = n_pad // NS
  zero_blocks = rows_per_tile // CHUNK
  mesh = plsc.VectorSubcoreMesh(core_axis_name="c", subcore_axis_name="s")

  out_type = [jax.ShapeDtypeStruct((NC, n_pad, D), jnp.float32)]
  if with_deg:
    out_type.append(jax.ShapeDtypeStruct((NC, n_pad), jnp.float32))

  scratch = [
      pltpu.VMEM((n_chunks, CHUNK), jnp.int32),
      pltpu.VMEM((n_chunks, CHUNK), jnp.int32),
      pltpu.VMEM((CHUNK, D), jnp.float32),
      pltpu.VMEM((CHUNK,), jnp.float32),
      pltpu.VMEM_SHARED((n_pad, D), jnp.float32),
      pltpu.VMEM_SHARED((n_pad,), jnp.float32),
      pltpu.SemaphoreType.DMA,
  ]

  @functools.partial(
      pl.kernel,
      mesh=mesh,
      out_type=tuple(out_type),
      scratch_types=scratch,
  )
  def sc_agg(x_hbm, src_hbm, dst_hbm, *refs):
    if with_deg:
      sum_out, deg_out = refs[0], refs[1]
      rest = refs[2:]
    else:
      sum_out = refs[0]
      deg_out = None
      rest = refs[1:]
    src_v, dst_v, rows_v, ones_v, acc_sh, deg_sh, sem = rest

    c = lax.axis_index("c")
    s = lax.axis_index("s")
    wid = s * NC + c
    base = s * rows_per_tile

    def zfill(i, _):
      rows_v[i // (D // 16), pl.ds((i % (D // 16)) * 16, 16)] = (
          jnp.zeros((16,), jnp.float32))
      return 0
    lax.fori_loop(0, CHUNK * (D // 16), zfill, 0)
    for i in range(CHUNK // 16):
      ones_v[pl.ds(i * 16, 16)] = jnp.ones((16,), jnp.float32)

    for k in range(zero_blocks):
      pltpu.sync_copy(rows_v, acc_sh.at[pl.ds(base + k * CHUNK, CHUNK)])
    if with_deg:
      for k in range(zero_blocks):
        pltpu.sync_copy(rows_v.at[0], deg_sh.at[pl.ds(base + k * CHUNK, CHUNK)])
    plsc.subcore_barrier()

    pltpu.sync_copy(src_hbm.at[wid], src_v)
    pltpu.sync_copy(dst_hbm.at[wid], dst_v)

    def body(i, _):
      pltpu.async_copy(x_hbm.at[src_v.at[i]], rows_v, sem).wait()
      pltpu.sync_copy(rows_v, acc_sh.at[dst_v.at[i]], add=True)
      if with_deg:
        pltpu.sync_copy(ones_v, deg_sh.at[dst_v.at[i]], add=True)
      return 0
    my_chunks = jnp.where(c == 0, k0, k1)
    lax.fori_loop(0, my_chunks, body, 0)

    plsc.subcore_barrier()
    pltpu.sync_copy(acc_sh.at[pl.ds(base, rows_per_tile)],
                    sum_out.at[c, pl.ds(base, rows_per_tile)])
    if with_deg:
      pltpu.sync_copy(deg_sh.at[pl.ds(base, rows_per_tile)],
                      deg_out.at[c, pl.ds(base, rows_per_tile)])

  return sc_agg


def _dot(a, b):
  return jnp.dot(a, b, precision=lax.Precision.HIGHEST,
                 preferred_element_type=jnp.float32)


def _leaky(h):
  return jnp.where(h >= 0, h, 0.01 * h)


def _tc_layer1(sums, deg, xp, Wl, bl, Wr, n_pad, br=512):
  def body(sum_ref, deg_ref, x_ref, wl_ref, bl_ref, wr_ref, h_ref):
    r = pl.program_id(0)
    s = sum_ref[0] + sum_ref[1]
    dg = deg_ref[:, pl.ds(r * br, br)]
    dg = jnp.clip(dg[0] + dg[1], 1.0, None)
    mean = s / dg[:, None]
    h = _dot(mean, wl_ref[...]) + bl_ref[...] + _dot(x_ref[...], wr_ref[...])
    norm = jnp.sqrt(jnp.sum(h * h, axis=1, keepdims=True))
    h = h / jnp.clip(norm, 1e-12, None)
    h_ref[...] = _leaky(h)

  return pl.pallas_call(
      body,
      grid=(n_pad // br,),
      in_specs=[
          pl.BlockSpec((NC, br, D), lambda r: (0, r, 0)),
          pl.BlockSpec((NC, n_pad), lambda r: (0, 0)),
          pl.BlockSpec((br, D), lambda r: (r, 0)),
          pl.BlockSpec((D, D), lambda r: (0, 0)),
          pl.BlockSpec((1, D), lambda r: (0, 0)),
          pl.BlockSpec((D, D), lambda r: (0, 0)),
      ],
      out_specs=pl.BlockSpec((br, D), lambda r: (r, 0)),
      out_shape=jax.ShapeDtypeStruct((n_pad, D), jnp.float32),
  )(sums, deg, xp, Wl, bl, Wr)


def _tc_layer2(sums, deg, hp, Wl, bl, Wr, Wlin, blin, n_pad, br=512):
  def body(sum_ref, deg_ref, h_ref, wl_ref, bl_ref, wr_ref, wlin_ref,
           blin_ref, out_ref):
    r = pl.program_id(0)
    s = sum_ref[0] + sum_ref[1]
    dg = deg_ref[:, pl.ds(r * br, br)]
    dg = jnp.clip(dg[0] + dg[1], 1.0, None)
    mean = s / dg[:, None]
    h = _dot(mean, wl_ref[...]) + bl_ref[...] + _dot(h_ref[...], wr_ref[...])
    h = _leaky(h)
    out_ref[...] = _dot(h, wlin_ref[...]) + blin_ref[...]

  return pl.pallas_call(
      body,
      grid=(n_pad // br,),
      in_specs=[
          pl.BlockSpec((NC, br, D), lambda r: (0, r, 0)),
          pl.BlockSpec((NC, n_pad), lambda r: (0, 0)),
          pl.BlockSpec((br, D), lambda r: (r, 0)),
          pl.BlockSpec((D, D), lambda r: (0, 0)),
          pl.BlockSpec((1, D), lambda r: (0, 0)),
          pl.BlockSpec((D, D), lambda r: (0, 0)),
          pl.BlockSpec((D, 1), lambda r: (0, 0)),
          pl.BlockSpec((1, 1), lambda r: (0, 0)),
      ],
      out_specs=pl.BlockSpec((br, 1), lambda r: (r, 0)),
      out_shape=jax.ShapeDtypeStruct((n_pad, 1), jnp.float32),
  )(sums, deg, hp, Wl, bl, Wr, Wlin, blin)


def kernel(x, edge_index, edge_weight, Wl1, bl1, Wr1, Wl2, bl2, Wr2,
           Wlin, blin):
  del edge_weight
  n = x.shape[0]
  e = edge_index.shape[1]

  rows_per_tile = -(-(n + 1) // (NS * CHUNK)) * CHUNK
  n_pad = NS * rows_per_tile

  pair_chunks = -(-e // (NS * CHUNK))
  k0 = int(round(pair_chunks * 0.645))
  k1 = pair_chunks - k0
  e_pad = NS * pair_chunks * CHUNK

  def to_worker_layout(flat):
    pool = flat.reshape(NS * pair_chunks, CHUNK)
    p0 = pool[:NS * k0].reshape(NS, k0, CHUNK)
    p1 = pool[NS * k0:].reshape(NS, k1, CHUNK)
    p1 = jnp.pad(p1, ((0, 0), (0, k0 - k1), (0, 0)))
    return jnp.stack([p0, p1], axis=1).reshape(NW, k0, CHUNK)

  pad_dst = n + jnp.arange(e_pad - e, dtype=jnp.int32) % (n_pad - n)
  src = to_worker_layout(jnp.concatenate(
      [edge_index[0], jnp.zeros((e_pad - e,), jnp.int32)]))
  dst = to_worker_layout(jnp.concatenate([edge_index[1], pad_dst]))

  xp = jnp.pad(x, ((0, n_pad - n), (0, 0)))

  sc_agg1 = _make_sc_agg(n_pad, k0, k1, with_deg=True)
  sc_agg2 = _make_sc_agg(n_pad, k0, k1, with_deg=False)

  sums1, deg = sc_agg1(xp, src, dst)
  h1 = _tc_layer1(sums1, deg, xp, Wl1, bl1.reshape(1, D), Wr1, n_pad)
  (sums2,) = sc_agg2(h1, src, dst)
  out = _tc_layer2(sums2, deg, h1, Wl2, bl2.reshape(1, D), Wr2,
                   Wlin, blin.reshape(1, 1), n_pad)
  return out[:n]


# speedup vs baseline: 1.6680x; 1.0681x over previous
import functools

import jax
import jax.numpy as jnp
from jax import lax
from jax.experimental import pallas as pl
from jax.experimental.pallas import tpu as pltpu
from jax.experimental.pallas import tpu_sc as plsc

D = 128
CHUNK = 128
NC = 2
NS = 16
NW = NC * NS


def _make_sc_agg(n_pad, k0, k1, with_deg):
  n_chunks = max(k0, k1)
  rows_per_tile = n_pad // NS
  zero_blocks = rows_per_tile // CHUNK
  mesh = plsc.VectorSubcoreMesh(core_axis_name="c", subcore_axis_name="s")

  out_type = [jax.ShapeDtypeStruct((NC, n_pad, D), jnp.float32)]
  if with_deg:
    out_type.append(jax.ShapeDtypeStruct((NC, n_pad), jnp.float32))

  scratch = [
      pltpu.VMEM((n_chunks, CHUNK), jnp.int32),
      pltpu.VMEM((n_chunks, CHUNK), jnp.int32),
      pltpu.VMEM((CHUNK, D), jnp.float32),
      pltpu.VMEM((CHUNK,), jnp.float32),
      pltpu.VMEM_SHARED((n_pad, D), jnp.float32),
      pltpu.VMEM_SHARED((n_pad,), jnp.float32),
      pltpu.SemaphoreType.DMA,
  ]

  @functools.partial(
      pl.kernel,
      mesh=mesh,
      out_type=tuple(out_type),
      scratch_types=scratch,
  )
  def sc_agg(x_hbm, src0_hbm, src1_hbm, dst0_hbm, dst1_hbm, *refs):
    if with_deg:
      sum_out, deg_out = refs[0], refs[1]
      rest = refs[2:]
    else:
      sum_out = refs[0]
      deg_out = None
      rest = refs[1:]
    src_v, dst_v, rows_v, ones_v, acc_sh, deg_sh, sem = rest

    c = lax.axis_index("c")
    s = lax.axis_index("s")
    wid = s * NC + c
    base = s * rows_per_tile

    def zfill(i, _):
      rows_v[i // (D // 16), pl.ds((i % (D // 16)) * 16, 16)] = (
          jnp.zeros((16,), jnp.float32))
      return 0
    lax.fori_loop(0, CHUNK * (D // 16), zfill, 0)
    for i in range(CHUNK // 16):
      ones_v[pl.ds(i * 16, 16)] = jnp.ones((16,), jnp.float32)

    for k in range(zero_blocks):
      pltpu.sync_copy(rows_v, acc_sh.at[pl.ds(base + k * CHUNK, CHUNK)])
    if with_deg:
      for k in range(zero_blocks):
        pltpu.sync_copy(rows_v.at[0], deg_sh.at[pl.ds(base + k * CHUNK, CHUNK)])
    plsc.subcore_barrier()

    @pl.when(c == 0)
    def _():
      pltpu.sync_copy(src0_hbm.at[s], src_v)
      pltpu.sync_copy(dst0_hbm.at[s], dst_v)
    @pl.when(c == 1)
    def _():
      pltpu.sync_copy(src1_hbm.at[s], src_v.at[pl.ds(0, k1)])
      pltpu.sync_copy(dst1_hbm.at[s], dst_v.at[pl.ds(0, k1)])

    def body(i, _):
      pltpu.async_copy(x_hbm.at[src_v.at[i]], rows_v, sem).wait()
      pltpu.sync_copy(rows_v, acc_sh.at[dst_v.at[i]], add=True)
      if with_deg:
        pltpu.sync_copy(ones_v, deg_sh.at[dst_v.at[i]], add=True)
      return 0
    my_chunks = jnp.where(c == 0, k0, k1)
    lax.fori_loop(0, my_chunks, body, 0)

    plsc.subcore_barrier()
    pltpu.sync_copy(acc_sh.at[pl.ds(base, rows_per_tile)],
                    sum_out.at[c, pl.ds(base, rows_per_tile)])
    if with_deg:
      pltpu.sync_copy(deg_sh.at[pl.ds(base, rows_per_tile)],
                      deg_out.at[c, pl.ds(base, rows_per_tile)])

  return sc_agg


def _dot(a, b):
  return jnp.dot(a, b, precision=lax.Precision.HIGHEST,
                 preferred_element_type=jnp.float32)


def _leaky(h):
  return jnp.where(h >= 0, h, 0.01 * h)


def _tc_layer1(sums, deg, xp, Wl, bl, Wr, n_pad, n, br):
  def body(sum_ref, deg_ref, x_ref, wl_ref, bl_ref, wr_ref, h_ref):
    s = sum_ref[0] + sum_ref[1]
    dg = jnp.clip(deg_ref[0, 0] + deg_ref[0, 1], 1.0, None)
    mean = s / dg[:, None]
    h = _dot(mean, wl_ref[...]) + bl_ref[...] + _dot(x_ref[...], wr_ref[...])
    norm = jnp.sqrt(jnp.sum(h * h, axis=1, keepdims=True))
    h = h / jnp.clip(norm, 1e-12, None)
    h_ref[...] = _leaky(h)

  return pl.pallas_call(
      body,
      grid=(n // br,),
      in_specs=[
          pl.BlockSpec((NC, br, D), lambda r: (0, r, 0)),
          pl.BlockSpec((1, NC, br), lambda r: (r, 0, 0)),
          pl.BlockSpec((br, D), lambda r: (r, 0)),
          pl.BlockSpec((D, D), lambda r: (0, 0)),
          pl.BlockSpec((1, D), lambda r: (0, 0)),
          pl.BlockSpec((D, D), lambda r: (0, 0)),
      ],
      out_specs=pl.BlockSpec((br, D), lambda r: (r, 0)),
      out_shape=jax.ShapeDtypeStruct((n, D), jnp.float32),
  )(sums, deg, xp, Wl, bl, Wr)


def _tc_layer2(sums, deg, hp, Wl, bl, Wr, Wlin, blin, n_pad, n, br):
  def body(sum_ref, deg_ref, h_ref, wl_ref, bl_ref, wr_ref, wlin_ref,
           blin_ref, out_ref):
    s = sum_ref[0] + sum_ref[1]
    dg = jnp.clip(deg_ref[0, 0] + deg_ref[0, 1], 1.0, None)
    mean = s / dg[:, None]
    h = _dot(mean, wl_ref[...]) + bl_ref[...] + _dot(h_ref[...], wr_ref[...])
    h = _leaky(h)
    out_ref[...] = _dot(h, wlin_ref[...]) + blin_ref[...]

  return pl.pallas_call(
      body,
      grid=(n // br,),
      in_specs=[
          pl.BlockSpec((NC, br, D), lambda r: (0, r, 0)),
          pl.BlockSpec((1, NC, br), lambda r: (r, 0, 0)),
          pl.BlockSpec((br, D), lambda r: (r, 0)),
          pl.BlockSpec((D, D), lambda r: (0, 0)),
          pl.BlockSpec((1, D), lambda r: (0, 0)),
          pl.BlockSpec((D, D), lambda r: (0, 0)),
          pl.BlockSpec((D, 1), lambda r: (0, 0)),
          pl.BlockSpec((1, 1), lambda r: (0, 0)),
      ],
      out_specs=pl.BlockSpec((br, 1), lambda r: (r, 0)),
      out_shape=jax.ShapeDtypeStruct((n, 1), jnp.float32),
  )(sums, deg, hp, Wl, bl, Wr, Wlin, blin)


def kernel(x, edge_index, edge_weight, Wl1, bl1, Wr1, Wl2, bl2, Wr2,
           Wlin, blin):
  del edge_weight
  n = x.shape[0]
  e = edge_index.shape[1]

  rows_per_tile = -(-(n + 1) // (NS * CHUNK)) * CHUNK
  n_pad = NS * rows_per_tile

  pair_chunks = -(-e // (NS * CHUNK))
  k0 = int(round(pair_chunks * 0.60))
  k1 = pair_chunks - k0
  e_pad = NS * pair_chunks * CHUNK

  def per_core_layout(flat):
    pool = flat.reshape(NS * pair_chunks, CHUNK)
    return (pool[:NS * k0].reshape(NS, k0, CHUNK),
            pool[NS * k0:].reshape(NS, k1, CHUNK))

  pad_dst = n + jnp.arange(e_pad - e, dtype=jnp.int32) % (n_pad - n)
  src0, src1 = per_core_layout(jnp.concatenate(
      [edge_index[0], jnp.zeros((e_pad - e,), jnp.int32)]))
  dst0, dst1 = per_core_layout(jnp.concatenate([edge_index[1], pad_dst]))

  sc_agg1 = _make_sc_agg(n_pad, k0, k1, with_deg=True)
  sc_agg2 = _make_sc_agg(n_pad, k0, k1, with_deg=False)

  br = next(b for b in (512, 400, 256, 200, 128, 80, 50, 40, 25, 20, 16,
                        10, 8, 5, 4, 2, 1) if n % b == 0)
  sums1, deg = sc_agg1(x, src0, src1, dst0, dst1)
  deg3 = deg[:, :n].reshape(NC, n // br, br).transpose(1, 0, 2)
  h1 = _tc_layer1(sums1, deg3, x, Wl1, bl1.reshape(1, D), Wr1, n_pad, n, br)
  (sums2,) = sc_agg2(h1, src0, src1, dst0, dst1)
  out = _tc_layer2(sums2, deg3, h1, Wl2, bl2.reshape(1, D), Wr2,
                   Wlin, blin.reshape(1, 1), n_pad, n, br)
  return out
